# baseline (speedup 1.0000x reference)
import jax
import jax.numpy as jnp
from jax.experimental import pallas as pl
from jax.experimental.pallas import tpu as pltpu

_EPS = 1e-12
_B = 8


def _netvlad_kernel(x_ref, w_ref, b_ref, c_ref, out_ref):
    w = w_ref[...]
    b = b_ref[...]
    cent = c_ref[...]

    for s in range(_B):
        xt = x_ref[s]
        xt16 = xt.astype(jnp.bfloat16)
        logits = jax.lax.dot_general(
            w.astype(jnp.bfloat16), xt16, (((1,), (1,)), ((), ())),
            preferred_element_type=jnp.float32) + b
        m = jnp.max(logits, axis=0, keepdims=True)
        e = jnp.exp(logits - m)
        a = e / jnp.sum(e, axis=0, keepdims=True)

        vlad = jax.lax.dot_general(
            a.astype(jnp.bfloat16), xt16, (((1,), (0,)), ((), ())),
            preferred_element_type=jnp.float32)
        vlad = vlad - jnp.sum(a, axis=1, keepdims=True) * cent

        inorm = jnp.sqrt(jnp.sum(vlad * vlad, axis=1, keepdims=True))
        vlad = vlad / jnp.maximum(inorm, _EPS)
        gnorm = jnp.sqrt(jnp.sum(vlad * vlad))
        out_ref[s] = vlad / jnp.maximum(gnorm, _EPS)


def kernel(x, conv_w, conv_b, centroids):
    N, C, H, W = x.shape
    K = centroids.shape[0]
    P = H * W
    xt = x.reshape(N, C, P).transpose(0, 2, 1)
    b2 = conv_b.reshape(K, 1)

    out = pl.pallas_call(
        _netvlad_kernel,
        grid=(N // _B,),
        in_specs=[
            pl.BlockSpec((_B, P, C), lambda n: (n, 0, 0)),
            pl.BlockSpec((K, C), lambda n: (0, 0)),
            pl.BlockSpec((K, 1), lambda n: (0, 0)),
            pl.BlockSpec((K, C), lambda n: (0, 0)),
        ],
        out_specs=pl.BlockSpec((_B, K, C), lambda n: (n, 0, 0)),
        out_shape=jax.ShapeDtypeStruct((N, K, C), jnp.float32),
        compiler_params=pltpu.CompilerParams(
            dimension_semantics=("parallel",),
            vmem_limit_bytes=56 * 1024 * 1024),
    )(xt, conv_w, b2, centroids)
    return out.reshape(N, K * C)


# speedup vs baseline: 1.4330x; 1.4330x over previous
import jax
import jax.numpy as jnp
from jax.experimental import pallas as pl
from jax.experimental.pallas import tpu as pltpu

_EPS = 1e-12
_B = 8


def _netvlad_kernel(x_ref, w_ref, b_ref, c_ref, out_ref):
    w = w_ref[...]
    b = b_ref[...]
    cent = c_ref[...]

    vlads = []
    for s in range(_B):
        xt = x_ref[s]
        xt16 = xt.astype(jnp.bfloat16)
        logits = jax.lax.dot_general(
            w.astype(jnp.bfloat16), xt16, (((1,), (1,)), ((), ())),
            preferred_element_type=jnp.float32) + b
        m = jnp.max(logits, axis=0, keepdims=True)
        e = jnp.exp(logits - m)
        a = e / jnp.sum(e, axis=0, keepdims=True)

        vlad = jax.lax.dot_general(
            a.astype(jnp.bfloat16), xt16, (((1,), (0,)), ((), ())),
            preferred_element_type=jnp.float32)
        vlad = vlad - jnp.sum(a, axis=1, keepdims=True) * cent

        inorm = jnp.sqrt(jnp.sum(vlad * vlad, axis=1, keepdims=True))
        vlad = vlad / jnp.maximum(inorm, _EPS)
        gnorm = jnp.sqrt(jnp.sum(vlad * vlad))
        vlads.append(vlad / jnp.maximum(gnorm, _EPS))
    K, C = w.shape
    out_ref[...] = jnp.stack(vlads, axis=0).reshape(_B, K * C)


def kernel(x, conv_w, conv_b, centroids):
    N, C, H, W = x.shape
    K = centroids.shape[0]
    P = H * W
    xt = x.reshape(N, C, P).transpose(0, 2, 1)
    b2 = conv_b.reshape(K, 1)

    out = pl.pallas_call(
        _netvlad_kernel,
        grid=(N // _B,),
        in_specs=[
            pl.BlockSpec((_B, P, C), lambda n: (n, 0, 0)),
            pl.BlockSpec((K, C), lambda n: (0, 0)),
            pl.BlockSpec((K, 1), lambda n: (0, 0)),
            pl.BlockSpec((K, C), lambda n: (0, 0)),
        ],
        out_specs=pl.BlockSpec((_B, K * C), lambda n: (n, 0)),
        out_shape=jax.ShapeDtypeStruct((N, K * C), jnp.float32),
        compiler_params=pltpu.CompilerParams(
            dimension_semantics=("parallel",),
            vmem_limit_bytes=56 * 1024 * 1024),
    )(xt, conv_w, b2, centroids)
    return out
